# baseline (speedup 1.0000x reference)
import jax
import jax.numpy as jnp
from jax.experimental import pallas as pl
from jax.experimental.pallas import tpu as pltpu

_B, _D, _K = 4096, 64, 8192
_BB = 256
_NSTEPS = _B // _BB
_NBUF = 4


def _copy(buf_ref, dist_ref, sem, step, slot):
    return pltpu.make_async_copy(
        buf_ref.at[slot],
        dist_ref.at[pl.ds(step * _BB, _BB), :],
        sem.at[slot])


def _body(x_ref, w_ref, dist_ref, imin_ref, buf_ref, wsq_ref, sem):
    b = pl.program_id(0)
    slot = jax.lax.rem(b, _NBUF)

    @pl.when(b == 0)
    def _():
        w0 = w_ref[...]
        wsq_ref[...] = jnp.sum(w0 * w0, axis=0, keepdims=True)

    @pl.when(b >= _NBUF)
    def _():
        _copy(buf_ref, dist_ref, sem, b - _NBUF, slot).wait()

    x = x_ref[...]
    xm2 = x * (-2.0)
    cross = jax.lax.dot_general(
        xm2, w_ref[...], (((1,), (0,)), ((), ())),
        preferred_element_type=jnp.float32)
    x_sq = jnp.sum(x * x, axis=1, keepdims=True)
    d = (x_sq + cross) + wsq_ref[...]
    buf_ref[slot] = d
    _copy(buf_ref, dist_ref, sem, b, slot).start()
    imin_ref[...] = jnp.argmin(d, axis=1).astype(jnp.int32)[:, None]

    @pl.when(b == _NSTEPS - 1)
    def _():
        for j in range(_NBUF):
            step_j = _NSTEPS - _NBUF + j
            _copy(buf_ref, dist_ref, sem, step_j, step_j % _NBUF).wait()


def kernel(x, w):
    dist, imin = pl.pallas_call(
        _body,
        grid=(_NSTEPS,),
        in_specs=[
            pl.BlockSpec((_BB, _D), lambda b: (b, 0)),
            pl.BlockSpec((_D, _K), lambda b: (0, 0)),
        ],
        out_specs=[
            pl.BlockSpec(memory_space=pl.ANY),
            pl.BlockSpec((_BB, 1), lambda b: (b, 0)),
        ],
        out_shape=[
            jax.ShapeDtypeStruct((_B, _K), jnp.float32),
            jax.ShapeDtypeStruct((_B, 1), jnp.int32),
        ],
        scratch_shapes=[
            pltpu.VMEM((_NBUF, _BB, _K), jnp.float32),
            pltpu.VMEM((1, _K), jnp.float32),
            pltpu.SemaphoreType.DMA((_NBUF,)),
        ],
    )(x, w)
    return dist, imin.reshape(_B)


# speedup vs baseline: 1.0078x; 1.0078x over previous
import jax
import jax.numpy as jnp
from jax.experimental import pallas as pl
from jax.experimental.pallas import tpu as pltpu

_B, _D, _K = 4096, 64, 8192
_BB = 256


def _body(x_ref, w_ref, dist_ref, imin_ref, wsq_ref):
    @pl.when(pl.program_id(0) == 0)
    def _():
        w0 = w_ref[...]
        wsq_ref[...] = jnp.sum(w0 * w0, axis=0, keepdims=True)

    x = x_ref[...]
    xm2 = x * (-2.0)
    cross = jax.lax.dot_general(
        xm2, w_ref[...], (((1,), (0,)), ((), ())),
        preferred_element_type=jnp.float32)
    x_sq = jnp.sum(x * x, axis=1, keepdims=True)
    d = (x_sq + cross) + wsq_ref[...]
    dist_ref[...] = d
    imin_ref[...] = jnp.argmin(d, axis=1).astype(jnp.int32)[:, None]


def kernel(x, w):
    dist, imin = pl.pallas_call(
        _body,
        grid=(_B // _BB,),
        in_specs=[
            pl.BlockSpec((_BB, _D), lambda b: (b, 0)),
            pl.BlockSpec((_D, _K), lambda b: (0, 0)),
        ],
        out_specs=[
            pl.BlockSpec((_BB, _K), lambda b: (b, 0)),
            pl.BlockSpec((_BB, 1), lambda b: (b, 0)),
        ],
        out_shape=[
            jax.ShapeDtypeStruct((_B, _K), jnp.float32),
            jax.ShapeDtypeStruct((_B, 1), jnp.int32),
        ],
        scratch_shapes=[pltpu.VMEM((1, _K), jnp.float32)],
    )(x, w)
    return dist, imin.reshape(_B)
